# baseline (speedup 1.0000x reference)
import jax
import jax.numpy as jnp
from jax import lax
from jax.experimental import pallas as pl
from jax.experimental.pallas import tpu as pltpu
from jax.experimental.pallas import tpu_sc as plsc

D = 128
NC = 2
NS = 16
L = 16
SUB = 2048
BATCH = 128
NCHUNK = 6
ZB = 32
PAD_DST = 1 << 30
IGN = -1


def _round_up(x, m):
    return (x + m - 1) // m * m


def _sc_gather(z, ei_flat):
    e_pad = ei_flat.shape[0] // 2
    ew = e_pad // NS
    eww = ew // NC
    nsub = eww // SUB

    mesh = plsc.VectorSubcoreMesh(core_axis_name="c", subcore_axis_name="s")

    def body(z_hbm, ei_hbm, msg_out, srct, gidx, rows, sem):
        c = lax.axis_index("c")
        s = lax.axis_index("s")
        wid = s * NC + c

        for t in range(nsub):
            off = wid * eww + t * SUB
            pltpu.sync_copy(ei_hbm.at[pl.ds(off, SUB)], srct)
            for bi in range(SUB // BATCH):
                def grp(g, _, bi=bi):
                    vs = srct[pl.ds(bi * BATCH + g * L, L)]
                    gidx[pl.ds(g * L, L)] = vs
                    return 0
                lax.fori_loop(0, BATCH // L, grp, 0)
                pltpu.async_copy(z_hbm.at[gidx], rows, sem).wait()
                pltpu.sync_copy(rows,
                                msg_out.at[pl.ds(off + bi * BATCH, BATCH)])

    fn = pl.kernel(body,
                   out_type=jax.ShapeDtypeStruct((e_pad, D), jnp.float32),
                   mesh=mesh,
                   scratch_types=[
                       pltpu.VMEM((SUB,), jnp.int32),
                       pltpu.VMEM((BATCH,), jnp.int32),
                       pltpu.VMEM((BATCH, D), jnp.float32),
                       pltpu.SemaphoreType.DMA,
                   ])
    return fn(z, ei_flat)


def _sc_agg(z, ei_flat, dst, n_dst, e_real):
    msg = _sc_gather(z, ei_flat)
    agg = jax.ops.segment_sum(msg[:e_real], dst, num_segments=n_dst)
    cnt = jax.ops.segment_sum(jnp.ones((e_real,), jnp.float32), dst,
                              num_segments=n_dst)
    cnt16 = jnp.broadcast_to(cnt[:, None], (n_dst, L))
    return agg, cnt16


_BM = 2000


def _mm_body(x_ref, w_ref, o_ref):
    o_ref[...] = lax.dot_general(
        x_ref[...], w_ref[...], (((1,), (1,)), ((), ())),
        preferred_element_type=jnp.float32)


def _mm(x, w):
    n = x.shape[0]
    return pl.pallas_call(
        _mm_body,
        grid=(n // _BM,),
        in_specs=[pl.BlockSpec((_BM, D), lambda i: (i, 0)),
                  pl.BlockSpec((D, D), lambda i: (0, 0))],
        out_specs=pl.BlockSpec((_BM, D), lambda i: (i, 0)),
        out_shape=jax.ShapeDtypeStruct((n, D), jnp.float32),
    )(x, w)


def _post1_body(agg_ref, cnt_ref, x_ref, wr_ref, b_ref, wn_ref,
                h_ref, z2_ref):
    inv = 1.0 / jnp.maximum(cnt_ref[:, 0:1], 1.0)
    pre = agg_ref[...] * inv + b_ref[...] + lax.dot_general(
        x_ref[...], wr_ref[...], (((1,), (1,)), ((), ())),
        preferred_element_type=jnp.float32)
    h = jnp.maximum(pre, 0.0)
    h_ref[...] = h
    z2_ref[...] = lax.dot_general(
        h, wn_ref[...], (((1,), (1,)), ((), ())),
        preferred_element_type=jnp.float32)


def _post1(agg_pad, cnt_pad, x_dst, wr, b, wnext):
    n = x_dst.shape[0]
    return pl.pallas_call(
        _post1_body,
        grid=(n // _BM,),
        in_specs=[pl.BlockSpec((_BM, D), lambda i: (i, 0)),
                  pl.BlockSpec((_BM, L), lambda i: (i, 0)),
                  pl.BlockSpec((_BM, D), lambda i: (i, 0)),
                  pl.BlockSpec((D, D), lambda i: (0, 0)),
                  pl.BlockSpec((1, D), lambda i: (0, 0)),
                  pl.BlockSpec((D, D), lambda i: (0, 0))],
        out_specs=[pl.BlockSpec((_BM, D), lambda i: (i, 0)),
                   pl.BlockSpec((_BM, D), lambda i: (i, 0))],
        out_shape=[jax.ShapeDtypeStruct((n, D), jnp.float32),
                   jax.ShapeDtypeStruct((n, D), jnp.float32)],
    )(agg_pad, cnt_pad, x_dst, wr, b, wnext)


def _post2_body(agg_ref, cnt_ref, x_ref, wr_ref, b_ref, o_ref):
    inv = 1.0 / jnp.maximum(cnt_ref[:, 0:1], 1.0)
    o_ref[...] = agg_ref[...] * inv + b_ref[...] + lax.dot_general(
        x_ref[...], wr_ref[...], (((1,), (1,)), ((), ())),
        preferred_element_type=jnp.float32)


def _post2(agg_pad, cnt_pad, x_dst, wr, b):
    n = x_dst.shape[0]
    return pl.pallas_call(
        _post2_body,
        grid=(n // _BM,),
        in_specs=[pl.BlockSpec((_BM, D), lambda i: (i, 0)),
                  pl.BlockSpec((_BM, L), lambda i: (i, 0)),
                  pl.BlockSpec((_BM, D), lambda i: (i, 0)),
                  pl.BlockSpec((D, D), lambda i: (0, 0)),
                  pl.BlockSpec((1, D), lambda i: (0, 0))],
        out_specs=pl.BlockSpec((_BM, D), lambda i: (i, 0)),
        out_shape=jax.ShapeDtypeStruct((n, D), jnp.float32),
    )(agg_pad, cnt_pad, x_dst, wr, b)


def _pad_edges(ei):
    e = ei.shape[1]
    ew = _round_up(_round_up(e, NS) // NS, SUB)
    e_pad = NS * ew
    pad = jnp.stack([jnp.zeros((e_pad - e,), jnp.int32),
                     jnp.full((e_pad - e,), PAD_DST, jnp.int32)])
    return jnp.concatenate([ei, pad], axis=1).reshape(-1)


def kernel(x_user, x_item, ei_u2i, ei_i2u,
           W1_u2i_l, b1_u2i, W1_u2i_r, W1_i2u_l, b1_i2u, W1_i2u_r,
           W2_u2i_l, b2_u2i, W2_u2i_r, W2_i2u_l, b2_i2u, W2_i2u_r):
    n_user = x_user.shape[0]
    n_item = x_item.shape[0]
    eu = _pad_edges(ei_u2i)
    ev = _pad_edges(ei_i2u)
    b1u = b1_u2i.reshape(1, D)
    b1i = b1_i2u.reshape(1, D)
    b2u = b2_u2i.reshape(1, D)
    b2i = b2_i2u.reshape(1, D)

    e_u = ei_u2i.shape[1]
    e_v = ei_i2u.shape[1]
    dst_u2i = ei_u2i[1]
    dst_i2u = ei_i2u[1]

    z1i = _mm(x_user, W1_u2i_l)
    z1u = _mm(x_item, W1_i2u_l)
    agg1i, cnt_i = _sc_agg(z1i, eu, dst_u2i, n_item, e_u)
    agg1u, cnt_u = _sc_agg(z1u, ev, dst_i2u, n_user, e_v)
    h_item, z2u = _post1(agg1i, cnt_i, x_item, W1_u2i_r, b1u, W2_i2u_l)
    h_user, z2i = _post1(agg1u, cnt_u, x_user, W1_i2u_r, b1i, W2_u2i_l)

    agg2i = _sc_agg(z2i, eu, dst_u2i, n_item, e_u)[0]
    agg2u = _sc_agg(z2u, ev, dst_i2u, n_user, e_v)[0]
    o_item = _post2(agg2i, cnt_i, h_item, W2_u2i_r, b2u)
    o_user = _post2(agg2u, cnt_u, h_user, W2_i2u_r, b2i)
    return (o_user, o_item)
